# Initial kernel scaffold; baseline (speedup 1.0000x reference)
import jax
import jax.numpy as jnp
from jax.experimental import pallas as pl


def kernel(left_x, left_graph_index, right_x, right_graph_index, left_x_batch, right_x_batch, specific_embedding, emb_diag, edges1_diag, edges2_diag, idx_diag, W_diag, asrc_diag, adst_diag, b_diag, emb_proce, edges1_proce, edges2_proce, idx_proce, W_proce, asrc_proce, adst_proce, b_proce, emb_atc, edges1_atc, edges2_atc, idx_atc, W_atc, asrc_atc, adst_atc, b_atc, W_g1, b_g1, W_g2, b_g2):
    raise NotImplementedError("write your pallas kernel here")


# speedup vs baseline: 4.8015x; 4.8015x over previous
import functools

import jax
import jax.numpy as jnp
from jax import lax
from jax.experimental import pallas as pl
from jax.experimental.pallas import tpu as pltpu
from jax.experimental.pallas import tpu_sc as plsc

F32 = jnp.float32
I32 = jnp.int32
D = 128
L = 16
NS = 16
NC = 2
CH = 128
BR = 512


def _leaky(x):
    return jnp.maximum(x, 0.0) + 0.2 * jnp.minimum(x, 0.0)


def _pad_rows(x, m):
    return jnp.pad(x, ((0, m - x.shape[0]), (0, 0)))


def _pad_vec(x, m, val=0):
    return jnp.pad(x, (0, m - x.shape[0]), constant_values=val)


@functools.partial(jax.jit, static_argnames=("pre",))
def _tc_gat(x, a1, s0, s1, b, W, asrc, adst, *, pre):
    N = x.shape[0]

    def body(x_ref, a1_ref, s0_ref, s1_ref, b_ref, w_ref, as_ref, ad_ref,
             h_ref, asv_ref, adv_ref, wsh_ref, hwh_ref):
        xv = x_ref[...]
        if pre:
            sden = s0_ref[...] + s1_ref[...]
            xv = (xv + a1_ref[...]) * (1.0 / sden)[:, None] + b_ref[...][None, :]
        h = jnp.dot(xv, w_ref[...], preferred_element_type=F32)
        asv = jnp.sum(h * as_ref[...][None, :], axis=1)
        adv = jnp.sum(h * ad_ref[...][None, :], axis=1)
        wsh = 0.5 * jnp.exp(_leaky(asv + adv))
        h_ref[...] = h
        asv_ref[...] = asv
        adv_ref[...] = adv
        wsh_ref[...] = wsh
        hwh_ref[...] = h * wsh[:, None]

    g = N // BR
    rspec = pl.BlockSpec((BR, D), lambda i: (i, 0))
    vspec = pl.BlockSpec((BR,), lambda i: (i,))
    cspec = pl.BlockSpec((D,), lambda i: (0,))
    wspec = pl.BlockSpec((D, D), lambda i: (0, 0))
    return pl.pallas_call(
        body,
        grid=(g,),
        in_specs=[rspec, rspec, vspec, vspec, cspec, wspec, cspec, cspec],
        out_specs=[rspec, vspec, vspec, vspec, rspec],
        out_shape=[
            jax.ShapeDtypeStruct((N, D), F32),
            jax.ShapeDtypeStruct((N,), F32),
            jax.ShapeDtypeStruct((N,), F32),
            jax.ShapeDtypeStruct((N,), F32),
            jax.ShapeDtypeStruct((N, D), F32),
        ],
    )(x, a1, s0, s1, b, W, asrc, adst)


@jax.jit
def _tc_fin(a0, a1, s0, s1, b):
    N = a0.shape[0]

    def body(a0_ref, a1_ref, s0_ref, s1_ref, b_ref, o_ref):
        sden = s0_ref[...] + s1_ref[...]
        o_ref[...] = ((a0_ref[...] + a1_ref[...]) * (1.0 / sden)[:, None]
                      + b_ref[...][None, :])

    rspec = pl.BlockSpec((BR, D), lambda i: (i, 0))
    vspec = pl.BlockSpec((BR,), lambda i: (i,))
    cspec = pl.BlockSpec((D,), lambda i: (0,))
    return pl.pallas_call(
        body,
        grid=(N // BR,),
        in_specs=[rspec, rspec, vspec, vspec, cspec],
        out_specs=rspec,
        out_shape=jax.ShapeDtypeStruct((N, D), F32),
    )(a0, a1, s0, s1, b)


@functools.partial(jax.jit, static_argnames=("pre",))
def _tc_gcn(x, a1, d0, d1, b, W, *, pre):
    N = x.shape[0]

    def body(x_ref, a1_ref, d0_ref, d1_ref, b_ref, w_ref, hd_ref, hdh_ref):
        dinv = lax.rsqrt(d0_ref[...] + d1_ref[...])
        xv = x_ref[...]
        if pre:
            xv = (xv + a1_ref[...]) * dinv[:, None] + b_ref[...][None, :]
        hd = jnp.dot(xv, w_ref[...], preferred_element_type=F32) * dinv[:, None]
        hd_ref[...] = hd
        hdh_ref[...] = 0.5 * hd

    rspec = pl.BlockSpec((BR, D), lambda i: (i, 0))
    vspec = pl.BlockSpec((BR,), lambda i: (i,))
    cspec = pl.BlockSpec((D,), lambda i: (0,))
    wspec = pl.BlockSpec((D, D), lambda i: (0, 0))
    return pl.pallas_call(
        body,
        grid=(N // BR,),
        in_specs=[rspec, rspec, vspec, vspec, cspec, wspec],
        out_specs=[rspec, rspec],
        out_shape=[jax.ShapeDtypeStruct((N, D), F32),
                   jax.ShapeDtypeStruct((N, D), F32)],
    )(x, a1, d0, d1, b, W)


@jax.jit
def _tc_cos(lr, ld, rr, rd, b):
    def body(lr_ref, ld_ref, rr_ref, rd_ref, b_ref, o_ref):
        bb = b_ref[...][None, :]
        lf = lr_ref[...] * lax.rsqrt(ld_ref[...])[:, None] + bb
        rf = rr_ref[...] * lax.rsqrt(rd_ref[...])[:, None] + bb
        n1 = jnp.sqrt(jnp.sum(lf * lf, axis=1))
        n2 = jnp.sqrt(jnp.sum(rf * rf, axis=1))
        o_ref[...] = (jnp.sum(lf * rf, axis=1)
                      / (jnp.maximum(n1, 1e-6) * jnp.maximum(n2, 1e-6)))

    return pl.pallas_call(
        body,
        out_shape=jax.ShapeDtypeStruct((512,), F32),
    )(lr, ld, rr, rd, b)


_SC_PARAMS = pltpu.CompilerParams(needs_layout_passes=False)


def _mesh(num_cores=None):
    if num_cores is None:
        return plsc.VectorSubcoreMesh(core_axis_name="c", subcore_axis_name="s")
    return plsc.VectorSubcoreMesh(core_axis_name="c", subcore_axis_name="s",
                                  num_cores=num_cores)


_COL16 = [None]


def _cols():
    return [lax.broadcasted_iota(I32, (L,), 0) + 16 * j for j in range(8)]


def _sc_gat_edge(n, n_up, NP, P, npass, E_pad):
    EPC = E_pad // NC
    EPT = EPC // NS
    NCHK = EPT // CH
    RPT = P // NS

    @functools.partial(
        pl.kernel,
        out_type=[jax.ShapeDtypeStruct((NC, n_up, D), F32),
                  jax.ShapeDtypeStruct((NC, 1, NP), F32)],
        mesh=_mesh(),
        compiler_params=_SC_PARAMS,
        scratch_types=[
            pltpu.VMEM_SHARED((P + 8, D), F32),
            pltpu.VMEM_SHARED((NP,), F32),
            pltpu.VMEM((EPT,), I32),
            pltpu.VMEM((EPT,), I32),
            pltpu.VMEM((CH,), F32),
            pltpu.VMEM((CH,), F32),
            pltpu.VMEM((EPT,), F32),
            pltpu.VMEM((CH, D), F32),
            pltpu.VMEM((1, CH), I32),
        ],
    )
    def k(h_hbm, hwh_hbm, asv_hbm, adv_hbm, wsh_hbm, src_hbm, dst_hbm,
          acc_out, s_out, sh_rows, sh_s, src_v, dst_v, a_v, b_v,
          wbuf, rows_v, idx2):
        c = lax.axis_index("c")
        t = lax.axis_index("s")
        toff = c * EPC + t * EPT
        pltpu.sync_copy(src_hbm.at[pl.ds(toff, EPT)], src_v)
        pltpu.sync_copy(dst_hbm.at[pl.ds(toff, EPT)], dst_v)

        @pl.when(t == 0)
        def _():
            pltpu.sync_copy(wsh_hbm, sh_s)
        plsc.subcore_barrier()
        for ch in range(NCHK):
            pltpu.sync_copy(asv_hbm.at[src_v.at[pl.ds(ch * CH, CH)]], a_v)
            pltpu.sync_copy(adv_hbm.at[dst_v.at[pl.ds(ch * CH, CH)]], b_v)
            for g in range(8):
                sl = pl.ds(g * L, L)
                esl = pl.ds(ch * CH + g * L, L)
                w = jnp.exp(_leaky(a_v[sl] + b_v[sl]))
                wbuf[esl] = w
                idx2[0, sl] = dst_v[esl]
            pltpu.sync_copy(wbuf.at[pl.ds(ch * CH, CH)],
                            sh_s.at[idx2.at[0]], add=True)
        plsc.subcore_barrier()

        @pl.when(t == 0)
        def _():
            pltpu.sync_copy(sh_s, s_out.at[c, 0])

        cols = _cols()
        for p in range(npass):
            base = p * P
            plsc.subcore_barrier()
            pltpu.sync_copy(hwh_hbm.at[pl.ds(base + t * RPT, RPT)],
                            sh_rows.at[pl.ds(t * RPT, RPT)])
            plsc.subcore_barrier()
            for ch in range(NCHK):
                pltpu.sync_copy(h_hbm.at[src_v.at[pl.ds(ch * CH, CH)]],
                                rows_v)

                def mul_body(e, _):
                    wv = plsc.load_gather(
                        wbuf, [jnp.full((L,), ch * CH + e, I32)])
                    er = jnp.full((L,), e, I32)
                    for j in range(8):
                        v = plsc.load_gather(rows_v, [er, cols[j]])
                        plsc.store_scatter(rows_v, [er, cols[j]], v * wv)
                    return 0

                lax.fori_loop(0, CH, mul_body, 0)
                for g in range(8):
                    dv = dst_v[pl.ds(ch * CH + g * L, L)]
                    inr = (dv >= base) & (dv < base + P)
                    idx2[0, pl.ds(g * L, L)] = jnp.where(inr, dv - base, P)
                pltpu.sync_copy(rows_v, sh_rows.at[idx2.at[0]], add=True)
            plsc.subcore_barrier()
            pltpu.sync_copy(sh_rows.at[pl.ds(t * RPT, RPT)],
                            acc_out.at[c, pl.ds(base + t * RPT, RPT)])

    return k


def _sc_gcn_edge(NU, E_pad):
    EPC = E_pad // NC
    EPT = EPC // NS
    NCHK = EPT // CH
    RPT = NU // NS

    @functools.partial(
        pl.kernel,
        out_type=jax.ShapeDtypeStruct((NC, NU, D), F32),
        mesh=_mesh(),
        compiler_params=_SC_PARAMS,
        scratch_types=[
            pltpu.VMEM_SHARED((NU, D), F32),
            pltpu.VMEM((EPT,), I32),
            pltpu.VMEM((EPT,), I32),
            pltpu.VMEM((CH, D), F32),
            pltpu.VMEM((1, CH), I32),
        ],
    )
    def k(hd_hbm, hdh_hbm, src_hbm, dst_hbm, acc_out,
          sh_rows, src_v, dst_v, rows_v, idx2):
        c = lax.axis_index("c")
        t = lax.axis_index("s")
        toff = c * EPC + t * EPT
        pltpu.sync_copy(src_hbm.at[pl.ds(toff, EPT)], src_v)
        pltpu.sync_copy(dst_hbm.at[pl.ds(toff, EPT)], dst_v)
        pltpu.sync_copy(hdh_hbm.at[pl.ds(t * RPT, RPT)],
                        sh_rows.at[pl.ds(t * RPT, RPT)])
        plsc.subcore_barrier()
        for ch in range(NCHK):
            pltpu.sync_copy(hd_hbm.at[src_v.at[pl.ds(ch * CH, CH)]], rows_v)
            for g in range(8):
                idx2[0, pl.ds(g * L, L)] = dst_v[pl.ds(ch * CH + g * L, L)]
            pltpu.sync_copy(rows_v, sh_rows.at[idx2.at[0]], add=True)
        plsc.subcore_barrier()
        pltpu.sync_copy(sh_rows.at[pl.ds(t * RPT, RPT)],
                        acc_out.at[c, pl.ds(t * RPT, RPT)])

    return k


def _sc_deg(n, NP, E_pad):
    EPC = E_pad // NC
    EPT = EPC // NS
    NCHK = EPT // CH

    @functools.partial(
        pl.kernel,
        out_type=jax.ShapeDtypeStruct((NC, 1, NP), F32),
        mesh=_mesh(),
        compiler_params=_SC_PARAMS,
        scratch_types=[
            pltpu.VMEM_SHARED((NP,), F32),
            pltpu.VMEM((EPT,), I32),
            pltpu.VMEM((CH,), F32),
            pltpu.VMEM((1, CH), I32),
        ],
    )
    def k(dst_hbm, halfones_hbm, s_out, sh_s, dst_v, ones_v, idx2):
        c = lax.axis_index("c")
        t = lax.axis_index("s")
        pltpu.sync_copy(dst_hbm.at[pl.ds(c * EPC + t * EPT, EPT)], dst_v)
        for g in range(8):
            ones_v[pl.ds(g * L, L)] = jnp.full((L,), 1.0, F32)

        @pl.when(t == 0)
        def _():
            pltpu.sync_copy(halfones_hbm, sh_s)
        plsc.subcore_barrier()
        for ch in range(NCHK):
            for g in range(8):
                idx2[0, pl.ds(g * L, L)] = dst_v[pl.ds(ch * CH + g * L, L)]
            pltpu.sync_copy(ones_v, sh_s.at[idx2.at[0]], add=True)
        plsc.subcore_barrier()

        @pl.when(t == 0)
        def _():
            pltpu.sync_copy(sh_s, s_out.at[c, 0])

    return k


def _sc_gather(T, B_pad):
    BPT = B_pad // (NC * NS)
    NCHK = BPT // CH

    @functools.partial(
        pl.kernel,
        out_type=jax.ShapeDtypeStruct((B_pad, D), F32),
        mesh=_mesh(),
        compiler_params=_SC_PARAMS,
        scratch_types=[
            pltpu.VMEM((BPT,), I32),
            pltpu.VMEM((CH, D), F32),
        ],
    )
    def k(tab_hbm, idx_hbm, out_hbm, idx_v, rows_v):
        c = lax.axis_index("c")
        t = lax.axis_index("s")
        boff = (c * NS + t) * BPT
        pltpu.sync_copy(idx_hbm.at[pl.ds(boff, BPT)], idx_v)
        for ch in range(NCHK):
            pltpu.sync_copy(tab_hbm.at[idx_v.at[pl.ds(ch * CH, CH)]], rows_v)
            pltpu.sync_copy(rows_v, out_hbm.at[pl.ds(boff + ch * CH, CH)])

    return k


def _sc_final(n, NP, BP):
    EPT = BP // NS
    NCHK = EPT // CH

    @functools.partial(
        pl.kernel,
        out_type=[jax.ShapeDtypeStruct((512, D), F32),
                  jax.ShapeDtypeStruct((512,), F32)],
        mesh=_mesh(num_cores=1),
        compiler_params=_SC_PARAMS,
        scratch_types=[
            pltpu.VMEM_SHARED((520,), I32),
            pltpu.VMEM((EPT,), I32),
            pltpu.VMEM((CH,), I32),
            pltpu.VMEM((1, CH), I32),
            pltpu.VMEM((512,), I32),
            pltpu.VMEM((512,), I32),
            pltpu.VMEM((NP,), F32),
            pltpu.VMEM((NP,), F32),
            pltpu.VMEM((CH, D), F32),
            pltpu.VMEM((CH, D), F32),
            pltpu.VMEM((512,), F32),
        ],
    )
    def k(batch_hbm, a0_hbm, a1_hbm, d0_hbm, d1_hbm, zeros_hbm,
          rows_out, dsel_out,
          sh_cnt, b_v, ones_v, idx2, cnt_v, cum_v, d0_v, d1_v,
          rows_v, rows2_v, dsel_v):
        t = lax.axis_index("s")
        pltpu.sync_copy(batch_hbm.at[pl.ds(t * EPT, EPT)], b_v)
        for g in range(8):
            ones_v[pl.ds(g * L, L)] = jnp.full((L,), 1, I32)

        @pl.when(t == 0)
        def _():
            pltpu.sync_copy(zeros_hbm, sh_cnt)
        plsc.subcore_barrier()
        for ch in range(NCHK):
            for g in range(8):
                idx2[0, pl.ds(g * L, L)] = b_v[pl.ds(ch * CH + g * L, L)]
            pltpu.sync_copy(ones_v, sh_cnt.at[idx2.at[0]], add=True)
        plsc.subcore_barrier()

        cols = _cols()

        @pl.when(t == 0)
        def _():
            pltpu.sync_copy(sh_cnt.at[pl.ds(0, 512)], cnt_v)

            def scan_body(g, carry):
                v = cnt_v[pl.ds(g * L, L)]
                inc = plsc.cumsum(v)
                ex = (inc - v) + jnp.full((L,), carry, I32)
                cum_v[pl.ds(g * L, L)] = jnp.minimum(ex, n - 1)
                return carry + inc[15]

            lax.fori_loop(0, 32, scan_body, jnp.int32(0))
            pltpu.sync_copy(d0_hbm, d0_v)
            pltpu.sync_copy(d1_hbm, d1_v)
            for ch in range(4):
                for g in range(8):
                    cv = cum_v[pl.ds(ch * CH + g * L, L)]
                    idx2[0, pl.ds(g * L, L)] = cv
                    dsel = (plsc.load_gather(d0_v, [cv])
                            + plsc.load_gather(d1_v, [cv]))
                    dsel_v[pl.ds(ch * CH + g * L, L)] = dsel
                pltpu.sync_copy(a0_hbm.at[idx2.at[0]], rows_v)
                pltpu.sync_copy(a1_hbm.at[idx2.at[0]], rows2_v)

                def add_body(e, _):
                    er = jnp.full((L,), e, I32)
                    for j in range(8):
                        v = (plsc.load_gather(rows_v, [er, cols[j]])
                             + plsc.load_gather(rows2_v, [er, cols[j]]))
                        plsc.store_scatter(rows_v, [er, cols[j]], v)
                    return 0

                lax.fori_loop(0, CH, add_body, 0)
                pltpu.sync_copy(rows_v, rows_out.at[pl.ds(ch * CH, CH)])
            pltpu.sync_copy(dsel_v, dsel_out)

    return k


_ONT_CFG = {
    40000: dict(n_up=40448, NP=40464, P=10112, npass=4, E_pad=40960, TCP=40960),
    30000: dict(n_up=30336, NP=30352, P=10112, npass=3, E_pad=32768, TCP=30720),
}


def _ontology(emb, e1, e2, idx, W, asrc, adst, b, v_out):
    n = emb.shape[0]
    cfg = _ONT_CFG[n]
    n_up, NP, P, npass, E_pad, TCP = (cfg["n_up"], cfg["NP"], cfg["P"],
                                      cfg["npass"], cfg["E_pad"], cfg["TCP"])
    gat_edge = _sc_gat_edge(n, n_up, NP, P, npass, E_pad)

    def pad_edges(e):
        src = _pad_vec(e[0].astype(I32), E_pad, 0)
        dst = _pad_vec(e[1].astype(I32), E_pad, n_up)
        return src, dst

    s1p, d1p = pad_edges(e1)
    s2p, d2p = pad_edges(e2)

    zeros_r = jnp.zeros((TCP, D), F32)
    zeros_v = jnp.zeros((TCP,), F32)
    ones_v = jnp.ones((TCP,), F32)

    h1, asv1, adv1, wsh1, hwh1 = _tc_gat(
        _pad_rows(emb, TCP), zeros_r, ones_v, zeros_v, b, W, asrc, adst,
        pre=False)
    acc1, s1 = gat_edge(h1[:n_up], hwh1[:n_up], asv1[:NP], adv1[:NP],
                        wsh1[:NP], s1p, d1p)
    h2, asv2, adv2, wsh2, hwh2 = _tc_gat(
        _pad_rows(acc1[0], TCP), _pad_rows(acc1[1], TCP),
        _pad_vec(s1[0, 0, :n], TCP, 1.0), _pad_vec(s1[1, 0, :n], TCP, 0.0),
        b, W, asrc, adst, pre=True)
    acc2, s2 = gat_edge(h2[:n_up], hwh2[:n_up], asv2[:NP], adv2[:NP],
                        wsh2[:NP], s2p, d2p)
    fin = _tc_fin(_pad_rows(acc2[0], TCP), _pad_rows(acc2[1], TCP),
                  _pad_vec(s2[0, 0, :n], TCP, 1.0),
                  _pad_vec(s2[1, 0, :n], TCP, 0.0), b)
    B_pad = {20000: 20480, 15000: 16384}[v_out]
    rows = _sc_gather(n, B_pad)(fin[:n], _pad_vec(idx.astype(I32), B_pad, 0))
    return rows[:v_out]


def _gcn_side(all_emb, x_idx, edges, batch, W1, b1, W2, b2):
    n = 10000
    NU = 10112
    NP = 10016
    E_pad = 323584
    TCP = 10240
    BP = 10240

    lx = _sc_gather(all_emb.shape[0], 12288)(
        all_emb, _pad_vec(x_idx.astype(I32), 12288, 0))[:n]

    src = _pad_vec(edges[0].astype(I32), E_pad, 0)
    dst = _pad_vec(edges[1].astype(I32), E_pad, n)

    degp = _sc_deg(n, NP, E_pad)(dst, jnp.full((NP,), 0.5, F32))
    d0 = _pad_vec(degp[0, 0, :n], TCP, 1.0)
    d1 = _pad_vec(degp[1, 0, :n], TCP, 0.0)

    zeros_r = jnp.zeros((TCP, D), F32)
    hd1, hdh1 = _tc_gcn(_pad_rows(lx, TCP), zeros_r, d0, d1, b1, W1,
                        pre=False)
    gcn_edge = _sc_gcn_edge(NU, E_pad)
    acc1 = gcn_edge(hd1[:NU], hdh1[:NU], src, dst)
    hd2, hdh2 = _tc_gcn(_pad_rows(acc1[0], TCP), _pad_rows(acc1[1], TCP),
                        d0, d1, b1, W2, pre=True)
    acc2 = gcn_edge(hd2[:NU], hdh2[:NU], src, dst)

    batch_p = _pad_vec(batch.astype(I32), BP, 512)
    rows, dsel = _sc_final(n, NP, BP)(
        batch_p, acc2[0], acc2[1], degp[0, 0], degp[1, 0],
        jnp.zeros((520,), I32))
    return rows, dsel


def kernel(left_x, left_graph_index, right_x, right_graph_index,
           left_x_batch, right_x_batch, specific_embedding,
           emb_diag, edges1_diag, edges2_diag, idx_diag, W_diag, asrc_diag,
           adst_diag, b_diag,
           emb_proce, edges1_proce, edges2_proce, idx_proce, W_proce,
           asrc_proce, adst_proce, b_proce,
           emb_atc, edges1_atc, edges2_atc, idx_atc, W_atc, asrc_atc,
           adst_atc, b_atc,
           W_g1, b_g1, W_g2, b_g2):
    diag = _ontology(emb_diag, edges1_diag, edges2_diag, idx_diag, W_diag,
                     asrc_diag, adst_diag, b_diag, 20000)
    proce = _ontology(emb_proce, edges1_proce, edges2_proce, idx_proce,
                      W_proce, asrc_proce, adst_proce, b_proce, 15000)
    atc = _ontology(emb_atc, edges1_atc, edges2_atc, idx_atc, W_atc,
                    asrc_atc, adst_atc, b_atc, 15000)
    all_emb = jnp.concatenate([specific_embedding, diag, proce, atc], axis=0)

    lrows, ldeg = _gcn_side(all_emb, left_x[:, 0], left_graph_index,
                            left_x_batch, W_g1, b_g1, W_g2, b_g2)
    rrows, rdeg = _gcn_side(all_emb, right_x[:, 0], right_graph_index,
                            right_x_batch, W_g1, b_g1, W_g2, b_g2)
    return _tc_cos(lrows, ldeg, rrows, rdeg, b_g2)
